# baseline (speedup 1.0000x reference)
import functools

import jax
import jax.numpy as jnp
from jax import lax
from jax.experimental import pallas as pl
from jax.experimental.pallas import tpu as pltpu
from jax.experimental.pallas import tpu_sc as plsc

VOCAB = 100000
D_IN = 768
D_OUT = 128
BATCH = 4096
SEQ = 50

NC = 2
NS = 16
NW = NC * NS
B_PER_W = BATCH // NW
GRP = 4
N_GRP = B_PER_W // GRP

ROWS_BLK = 2000


def _proj_body(t_ref, w_ref, b_ref, o_ref):
    o_ref[...] = (
        jnp.dot(t_ref[...], w_ref[...], preferred_element_type=jnp.float32)
        + b_ref[...]
    )


def _project_table(table, W, b2d):
    return pl.pallas_call(
        _proj_body,
        grid=(VOCAB // ROWS_BLK,),
        in_specs=[
            pl.BlockSpec((ROWS_BLK, D_IN), lambda i: (i, 0)),
            pl.BlockSpec((D_IN, D_OUT), lambda i: (0, 0)),
            pl.BlockSpec((1, D_OUT), lambda i: (0, 0)),
        ],
        out_specs=pl.BlockSpec((ROWS_BLK, D_OUT), lambda i: (i, 0)),
        out_shape=jax.ShapeDtypeStruct((VOCAB, D_OUT), jnp.float32),
    )(table, W, b2d)


def _gather_body(pt_hbm, idx_hbm, out_hbm, idx_v, rows_v, gsem0, gsem1,
                 wsem0, wsem1):
    gsems = (gsem0, gsem1)
    wsems = (wsem0, wsem1)
    wid = lax.axis_index("s") * NC + lax.axis_index("c")
    b0 = wid * B_PER_W
    pltpu.sync_copy(idx_hbm.at[pl.ds(b0, B_PER_W)], idx_v)

    def _write_desc(g, p):
        return pltpu.make_async_copy(
            rows_v.at[pl.ds(p * GRP, GRP)],
            out_hbm.at[pl.ds(b0 + g * GRP, GRP)],
            wsems[p],
        )

    def body(i, carry):
        gathers = []
        for p in range(2):
            g = 2 * i + p

            @pl.when(i > 0)
            def _():
                _write_desc(g - 2, p).wait()

            for j in range(GRP):
                gathers.append(pltpu.async_copy(
                    pt_hbm.at[idx_v.at[g * GRP + j]],
                    rows_v.at[p * GRP + j],
                    gsems[p],
                ))
        for p in range(2):
            g = 2 * i + p
            for j in range(GRP):
                gathers[p * GRP + j].wait()
            _write_desc(g, p).start()
        return carry

    lax.fori_loop(0, N_GRP // 2, body, 0)
    _write_desc(N_GRP - 2, 0).wait()
    _write_desc(N_GRP - 1, 1).wait()


def _gather(pt, idx):
    mesh = plsc.VectorSubcoreMesh(
        core_axis_name="c", subcore_axis_name="s", num_cores=NC, num_subcores=NS
    )
    k = functools.partial(
        pl.kernel,
        out_type=jax.ShapeDtypeStruct((BATCH, SEQ, D_OUT), jnp.float32),
        mesh=mesh,
        compiler_params=pltpu.CompilerParams(use_tc_tiling_on_sc=True),
        scratch_types=[
            pltpu.VMEM((B_PER_W, SEQ), jnp.int32),
            pltpu.VMEM((2 * GRP, SEQ, D_OUT), jnp.float32),
            pltpu.SemaphoreType.DMA,
            pltpu.SemaphoreType.DMA,
            pltpu.SemaphoreType.DMA,
            pltpu.SemaphoreType.DMA,
        ],
    )(_gather_body)
    return k(pt, idx)


def kernel(x, table, W, b):
    pt = _project_table(table, W, b.reshape(1, D_OUT))
    return _gather(pt, x.astype(jnp.int32))


# speedup vs baseline: 1.3218x; 1.3218x over previous
import functools

import jax
import jax.numpy as jnp
from jax import lax
from jax.experimental import pallas as pl
from jax.experimental.pallas import tpu as pltpu
from jax.experimental.pallas import tpu_sc as plsc

VOCAB = 100000
D_IN = 768
D_OUT = 128
BATCH = 4096
SEQ = 50

NC = 2
NS = 16
NW = NC * NS
B_PER_W = BATCH // NW

ROWS_BLK = 2000


def _proj_body(t_ref, w_ref, b_ref, o_ref):
    o_ref[...] = (
        jnp.dot(t_ref[...], w_ref[...], preferred_element_type=jnp.float32)
        + b_ref[...]
    )


def _project_table(table, W, b2d):
    return pl.pallas_call(
        _proj_body,
        grid=(VOCAB // ROWS_BLK,),
        in_specs=[
            pl.BlockSpec((ROWS_BLK, D_IN), lambda i: (i, 0)),
            pl.BlockSpec((D_IN, D_OUT), lambda i: (0, 0)),
            pl.BlockSpec((1, D_OUT), lambda i: (0, 0)),
        ],
        out_specs=pl.BlockSpec((ROWS_BLK, D_OUT), lambda i: (i, 0)),
        out_shape=jax.ShapeDtypeStruct((VOCAB, D_OUT), jnp.float32),
    )(table, W, b2d)


def _gather_body(pt_hbm, idxt_hbm, out_hbm, idx_v, rows_v, gsem0, gsem1,
                 wsem0, wsem1):
    gsems = (gsem0, gsem1)
    wsems = (wsem0, wsem1)
    wid = lax.axis_index("s") * NC + lax.axis_index("c")
    b0 = wid * B_PER_W
    pltpu.sync_copy(idxt_hbm.at[:, pl.ds(b0, B_PER_W)], idx_v)

    def _write_desc(s, p):
        return pltpu.make_async_copy(
            rows_v.at[p], out_hbm.at[s, pl.ds(b0, B_PER_W)], wsems[p]
        )

    def body(i, carry):
        gathers = []
        for p in range(2):
            s = 2 * i + p

            @pl.when(i > 0)
            def _():
                _write_desc(s - 2, p).wait()

            gathers.append(pltpu.async_copy(
                pt_hbm.at[idx_v.at[s]], rows_v.at[p], gsems[p]
            ))
        for p in range(2):
            s = 2 * i + p
            gathers[p].wait()
            _write_desc(s, p).start()
        return carry

    lax.fori_loop(0, SEQ // 2, body, 0)
    _write_desc(SEQ - 2, 0).wait()
    _write_desc(SEQ - 1, 1).wait()


def _gather(pt, idxt):
    mesh = plsc.VectorSubcoreMesh(
        core_axis_name="c", subcore_axis_name="s", num_cores=NC, num_subcores=NS
    )
    k = functools.partial(
        pl.kernel,
        out_type=jax.ShapeDtypeStruct((SEQ, BATCH, D_OUT), jnp.float32),
        mesh=mesh,
        scratch_types=[
            pltpu.VMEM((SEQ, B_PER_W), jnp.int32),
            pltpu.VMEM((2, B_PER_W, D_OUT), jnp.float32),
            pltpu.SemaphoreType.DMA,
            pltpu.SemaphoreType.DMA,
            pltpu.SemaphoreType.DMA,
            pltpu.SemaphoreType.DMA,
        ],
    )(_gather_body)
    return k(pt, idxt)


def kernel(x, table, W, b):
    pt = _project_table(table, W, b.reshape(1, D_OUT))
    idxt = jnp.swapaxes(x.astype(jnp.int32), 0, 1)
    out_sm = _gather(pt, idxt)
    return jnp.swapaxes(out_sm, 0, 1)


# speedup vs baseline: 1.4027x; 1.0613x over previous
import functools

import jax
import jax.numpy as jnp
from jax import lax
from jax.experimental import pallas as pl
from jax.experimental.pallas import tpu as pltpu
from jax.experimental.pallas import tpu_sc as plsc

VOCAB = 100000
D_IN = 768
D_OUT = 128
BATCH = 4096
SEQ = 50

NC = 2
NS = 16
NW = NC * NS
B_PER_W = BATCH // NW

ROWS_BLK = 5000


def _proj_body(t_ref, w_ref, b_ref, o_ref):
    o_ref[...] = (
        jnp.dot(t_ref[...], w_ref[...], preferred_element_type=jnp.float32)
        + b_ref[...]
    )


def _project_table(table, W, b2d):
    return pl.pallas_call(
        _proj_body,
        grid=(VOCAB // ROWS_BLK,),
        in_specs=[
            pl.BlockSpec((ROWS_BLK, D_IN), lambda i: (i, 0)),
            pl.BlockSpec((D_IN, D_OUT), lambda i: (0, 0)),
            pl.BlockSpec((1, D_OUT), lambda i: (0, 0)),
        ],
        out_specs=pl.BlockSpec((ROWS_BLK, D_OUT), lambda i: (i, 0)),
        out_shape=jax.ShapeDtypeStruct((VOCAB, D_OUT), jnp.float32),
    )(table, W, b2d)


NBUF = 5
N_MAIN = (SEQ // NBUF) * NBUF


def _gather_body(pt_hbm, idxt_hbm, out_hbm, idx_v, rows_v, *sems):
    gsems = sems[:NBUF]
    wsems = sems[NBUF:]
    wid = lax.axis_index("s") * NC + lax.axis_index("c")
    b0 = wid * B_PER_W
    pltpu.sync_copy(idxt_hbm.at[:, pl.ds(b0, B_PER_W)], idx_v)

    def _write_desc(s, p):
        return pltpu.make_async_copy(
            rows_v.at[p], out_hbm.at[s, pl.ds(b0, B_PER_W)], wsems[p]
        )

    def body(i, carry):
        gathers = []
        for p in range(NBUF):
            s = NBUF * i + p

            @pl.when(i > 0)
            def _():
                _write_desc(s - NBUF, p).wait()

            gathers.append(pltpu.async_copy(
                pt_hbm.at[idx_v.at[s]], rows_v.at[p], gsems[p]
            ))
        for p in range(NBUF):
            s = NBUF * i + p
            gathers[p].wait()
            _write_desc(s, p).start()
        return carry

    lax.fori_loop(0, SEQ // NBUF, body, 0)
    for p in range(NBUF):
        _write_desc(N_MAIN - NBUF + p, p).wait()
    for s in range(N_MAIN, SEQ):
        p = s - N_MAIN
        pltpu.async_copy(
            pt_hbm.at[idx_v.at[s]], rows_v.at[p], gsems[p]
        ).wait()
        _write_desc(s, p).start()
    for s in range(N_MAIN, SEQ):
        _write_desc(s, s - N_MAIN).wait()


def _gather(pt, idxt):
    mesh = plsc.VectorSubcoreMesh(
        core_axis_name="c", subcore_axis_name="s", num_cores=NC, num_subcores=NS
    )
    k = functools.partial(
        pl.kernel,
        out_type=jax.ShapeDtypeStruct((SEQ, BATCH, D_OUT), jnp.float32),
        mesh=mesh,
        scratch_types=[
            pltpu.VMEM((SEQ, B_PER_W), jnp.int32),
            pltpu.VMEM((NBUF, B_PER_W, D_OUT), jnp.float32),
        ] + [pltpu.SemaphoreType.DMA] * (2 * NBUF),
    )(_gather_body)
    return k(pt, idxt)


def kernel(x, table, W, b):
    pt = _project_table(table, W, b.reshape(1, D_OUT))
    idxt = jnp.swapaxes(x.astype(jnp.int32), 0, 1)
    out_sm = _gather(pt, idxt)
    return jnp.swapaxes(out_sm, 0, 1)
